# baseline (speedup 1.0000x reference)
import jax
import jax.numpy as jnp
from jax import lax
from jax.experimental import pallas as pl
from jax.experimental.pallas import tpu as pltpu
from jax.experimental.pallas import tpu_sc as plsc

N = 10000
D = 128
E = 320000

NC = 2
NS = 16
L = 16
NW = NC * NS
NPAD = 10240
NPT = NPAD // NS

K = 40
SUP = 10
EPT = E // NW
CHUNKS = EPT // K
NSUP = CHUNKS // SUP

ROWS_TC = 2000


def _prep_body(h_ref, wm_ref, bm_ref, wu_ref, bu_ref, a_ref, b_ref, c_ref):
    h = h_ref[...]
    a_ref[...] = jnp.dot(h, wm_ref[0:D, :], preferred_element_type=jnp.float32)
    b_ref[...] = (
        jnp.dot(h, wm_ref[D:, :], preferred_element_type=jnp.float32)
        + bm_ref[...]
    )
    c_ref[...] = (
        jnp.dot(h, wu_ref[0:D, :], preferred_element_type=jnp.float32)
        + bu_ref[...]
    )


def _prep(h, w_msg, b_msg, w_upd, b_upd):
    n = h.shape[0]
    grid = (n // ROWS_TC,)
    row_spec = pl.BlockSpec((ROWS_TC, D), lambda i: (i, 0))
    full_w = pl.BlockSpec((2 * D, D), lambda i: (0, 0))
    full_b = pl.BlockSpec((1, D), lambda i: (0, 0))
    out = jax.ShapeDtypeStruct((n, D), jnp.float32)
    return pl.pallas_call(
        _prep_body,
        grid=grid,
        in_specs=[row_spec, full_w, full_b, full_w, full_b],
        out_specs=[row_spec, row_spec, row_spec],
        out_shape=[out, out, out],
    )(h, w_msg, b_msg.reshape(1, D), w_upd, b_upd.reshape(1, D))


def _sc_body(a_hbm, b_hbm, src_hbm, tgt_hbm, out_msg, cstage,
             acc,
             isrc, itgt, itg2,
             stgt0, stgt1,
             ar0, ar1, br0, br1, or0, or1,
             cnt_local, cbuf,
             sem_a, sem_b, sem_o0, sem_o1, sem_i, sem_s):
    c = lax.axis_index("c")
    s = lax.axis_index("s")
    wid = s * NC + c
    wid2 = s * NC + (1 - c)
    row0 = s * NPT
    e0 = wid * EPT
    e02 = wid2 * EPT

    stgt = (stgt0, stgt1)
    arows = (ar0, ar1)
    brows = (br0, br1)
    orows = (or0, or1)
    sem_o = (sem_o0, sem_o1)

    zvec = jnp.zeros((L,), jnp.float32)
    ones16 = jnp.ones((L,), jnp.float32)
    tail_mask = lax.broadcasted_iota(jnp.int32, (L,), 0) >= (3 * L - K)

    @pl.loop(0, K)
    def _zero_rows(r):
        for j in range(D // L):
            or0[r, pl.ds(j * L, L)] = zvec

    @pl.loop(0, NPAD // L)
    def _zero_cnt(i):
        cnt_local[pl.ds(i * L, L)] = zvec

    for k in range(NPT // K):
        pltpu.sync_copy(or0, acc.at[pl.ds(row0 + k * K, K)])
    plsc.subcore_barrier()

    def issue_gathers(j, p):
        da = pltpu.async_copy(a_hbm.at[isrc.at[pl.ds(j * K, K)]], arows[p],
                              sem_a)
        db = pltpu.async_copy(b_hbm.at[itgt.at[pl.ds(j * K, K)]], brows[p],
                              sem_b)
        return da, db

    def compute(p):
        @pl.loop(0, K)
        def _rows(r):
            for j in range(D // L):
                va = arows[p][r, pl.ds(j * L, L)]
                vb = brows[p][r, pl.ds(j * L, L)]
                orows[p][r, pl.ds(j * L, L)] = jnp.maximum(va + vb, 0.0)

    def counts(j):
        for ref in (itgt, itg2):
            for g in range(2):
                plsc.addupdate_scatter(
                    cnt_local, [ref[pl.ds(j * K + g * L, L)]], ones16)
            plsc.addupdate_scatter(
                cnt_local, [ref[pl.ds(j * K + K - L, L)]], ones16,
                mask=tail_mask)

    @pl.loop(0, NSUP)
    def _super(su):
        base = su * SUP * K
        di1 = pltpu.async_copy(src_hbm.at[pl.ds(e0 + base, SUP * K)],
                               isrc, sem_i)
        di2 = pltpu.async_copy(tgt_hbm.at[pl.ds(e0 + base, SUP * K)],
                               itgt, sem_i)
        di3 = pltpu.async_copy(tgt_hbm.at[pl.ds(e02 + base, SUP * K)],
                               itg2, sem_i)
        di1.wait()
        di2.wait()
        di3.wait()

        dg = issue_gathers(0, 0)
        dsc = None
        for j in range(SUP):
            p = j % 2
            dst = pltpu.async_copy(
                tgt_hbm.at[pl.ds(e0 + base + j * K, K)], stgt[p], sem_s)
            dg[0].wait()
            dg[1].wait()
            if j < SUP - 1:
                dg_next = issue_gathers(j + 1, 1 - p)
            compute(p)
            counts(j)
            if dsc is not None:
                dsc.wait()
            dst.wait()
            dsc = pltpu.async_copy(orows[p], acc.at[stgt[p]], sem_o[p],
                                   add=True)
            if j < SUP - 1:
                dg = dg_next
        dsc.wait()

    for o in range(NS):
        pltpu.sync_copy(cnt_local.at[pl.ds(o * NPT, NPT)], cstage.at[c, o, s])
    plsc.subcore_barrier()

    @pl.loop(0, NPT // L)
    def _zero_sum(j):
        cnt_local[pl.ds(j * L, L)] = zvec

    for t in range(NS):
        pltpu.sync_copy(cstage.at[c, s, t], cbuf)

        @pl.loop(0, NPT // L)
        def _accum(j):
            cnt_local[pl.ds(j * L, L)] = (
                cnt_local[pl.ds(j * L, L)] + cbuf[pl.ds(j * L, L)]
            )

    @pl.loop(0, NPT // L)
    def _recip(j):
        cnt_local[pl.ds(j * L, L)] = 1.0 / jnp.maximum(
            cnt_local[pl.ds(j * L, L)], 1.0
        )

    for k in range(NPT // K):
        r = row0 + k * K
        pltpu.sync_copy(acc.at[pl.ds(r, K)], or0)

        @pl.loop(0, K)
        def _scale(q):
            rec = plsc.load_gather(cnt_local, [jnp.full((L,), k * K + q,
                                                        jnp.int32)])
            for j in range(D // L):
                or0[q, pl.ds(j * L, L)] = or0[q, pl.ds(j * L, L)] * rec

        pltpu.sync_copy(or0, out_msg.at[c, pl.ds(r, K)])


def _sc_scatter(a, b, src, tgt):
    mesh = plsc.VectorSubcoreMesh(core_axis_name="c", subcore_axis_name="s")
    f = pl.kernel(
        _sc_body,
        out_type=(
            jax.ShapeDtypeStruct((NC, NPAD, D), jnp.float32),
            jax.ShapeDtypeStruct((NC, NS, NS, NPT), jnp.float32),
        ),
        mesh=mesh,
        scratch_types=[
            pltpu.VMEM_SHARED((NPAD, D), jnp.float32),
            pltpu.VMEM((SUP * K,), jnp.int32),
            pltpu.VMEM((SUP * K,), jnp.int32),
            pltpu.VMEM((SUP * K,), jnp.int32),
            pltpu.VMEM((K,), jnp.int32),
            pltpu.VMEM((K,), jnp.int32),
            pltpu.VMEM((K, D), jnp.float32),
            pltpu.VMEM((K, D), jnp.float32),
            pltpu.VMEM((K, D), jnp.float32),
            pltpu.VMEM((K, D), jnp.float32),
            pltpu.VMEM((K, D), jnp.float32),
            pltpu.VMEM((K, D), jnp.float32),
            pltpu.VMEM((NPAD,), jnp.float32),
            pltpu.VMEM((NPT,), jnp.float32),
            pltpu.SemaphoreType.DMA,
            pltpu.SemaphoreType.DMA,
            pltpu.SemaphoreType.DMA,
            pltpu.SemaphoreType.DMA,
            pltpu.SemaphoreType.DMA,
            pltpu.SemaphoreType.DMA,
        ],
        compiler_params=pltpu.CompilerParams(needs_layout_passes=False),
    )
    return f(a, b, src, tgt)


def _final_body(h_ref, c_ref, parts_ref, wu_ref, g_ref, be_ref, out_ref):
    messages = parts_ref[0] + parts_ref[1]
    upd = jnp.maximum(
        c_ref[...]
        + jnp.dot(messages, wu_ref[D:, :], preferred_element_type=jnp.float32),
        0.0,
    )
    h2 = upd + h_ref[...]
    mu = jnp.mean(h2, axis=-1, keepdims=True)
    zc = h2 - mu
    var = jnp.mean(zc * zc, axis=-1, keepdims=True)
    out_ref[...] = zc * lax.rsqrt(var + 1e-5) * g_ref[...] + be_ref[...]


def _final(h, c, parts, w_upd, gamma, beta):
    n = h.shape[0]
    grid = (n // ROWS_TC,)
    row_spec = pl.BlockSpec((ROWS_TC, D), lambda i: (i, 0))
    parts_spec = pl.BlockSpec((NC, ROWS_TC, D), lambda i: (0, i, 0))
    full_w = pl.BlockSpec((2 * D, D), lambda i: (0, 0))
    full_b = pl.BlockSpec((1, D), lambda i: (0, 0))
    return pl.pallas_call(
        _final_body,
        grid=grid,
        in_specs=[row_spec, row_spec, parts_spec, full_w, full_b, full_b],
        out_specs=row_spec,
        out_shape=jax.ShapeDtypeStruct((n, D), jnp.float32),
    )(h, c, parts, w_upd, gamma.reshape(1, D), beta.reshape(1, D))


def kernel(node_features, adjacency, W_msg, b_msg, W_upd, b_upd, gamma, beta):
    src = adjacency[:, 0]
    tgt = adjacency[:, 1]
    a, b, c = _prep(node_features, W_msg, b_msg, W_upd, b_upd)
    parts, _ = _sc_scatter(a, b, src, tgt)
    return _final(node_features, c, parts, W_upd, gamma, beta)
